# baseline (speedup 1.0000x reference)
import functools

import jax
import jax.numpy as jnp
from jax import lax
from jax.experimental import pallas as pl
from jax.experimental.pallas import tpu as pltpu
from jax.experimental.pallas import tpu_sc as plsc

_VOCAB = 1000
_DIM = 64
_TOTAL = 16384 * 200
_NC, _NS = 2, 16
_NW = _NC * _NS
_PER_W = _TOTAL // _NW
_CHUNK = 400
_NCH = _PER_W // _CHUNK
_UNROLL = 16


def _emb_body(x_hbm, w_hbm, out_hbm, table_v, rows0, rows1,
              idx_s0, idx_s1, sem_i0, sem_i1, sem_s0, sem_s1):
    wid = lax.axis_index("s") * _NC + lax.axis_index("c")
    base = wid * _PER_W

    pltpu.sync_copy(w_hbm, table_v)

    iota = lax.iota(jnp.int32, 16)
    rows_b = (rows0, rows1)
    idx_b = (idx_s0, idx_s1)
    sem_i = (sem_i0, sem_i1)
    sem_s = (sem_s0, sem_s1)

    def idx_sl(ci):
        return x_hbm.at[pl.ds(base + ci * _CHUNK, _CHUNK)]

    def out_sl(ci):
        return out_hbm.at[pl.ds((base + ci * _CHUNK) * _DIM, _CHUNK * _DIM)]

    def compute(b):
        idx_s, rows = idx_b[b], rows_b[b]

        def grp(j, carry):
            j0 = j * _UNROLL
            idx_vec = idx_s[pl.ds(j0, _UNROLL)]
            for u in range(_UNROLL):
                src0 = idx_vec[u] * _DIM
                dst0 = (j0 + u) * _DIM
                for c in range(0, _DIM, 16):
                    vals = plsc.load_gather(table_v, [src0 + c + iota])
                    rows[pl.ds(dst0 + c, 16)] = vals
            return carry

        lax.fori_loop(0, _CHUNK // _UNROLL, grp, 0)

    h0 = pltpu.async_copy(idx_sl(0), idx_s0, sem_i0)
    h1 = pltpu.async_copy(idx_sl(1), idx_s1, sem_i1)
    for b, h in ((0, h0), (1, h1)):
        h.wait()
        compute(b)
        pltpu.async_copy(rows_b[b], out_sl(b), sem_s[b])
        pltpu.async_copy(idx_sl(b + 2), idx_b[b], sem_i[b])

    def steady(k, carry):
        ci2 = 2 + 2 * k
        for b in range(2):
            ci = ci2 + b
            pltpu.make_async_copy(idx_sl(ci), idx_b[b], sem_i[b]).wait()
            pltpu.make_async_copy(rows_b[b], out_sl(ci), sem_s[b]).wait()
            compute(b)
            pltpu.async_copy(rows_b[b], out_sl(ci), sem_s[b])
            pltpu.async_copy(idx_sl(ci + 2), idx_b[b], sem_i[b])
        return carry

    lax.fori_loop(0, (_NCH - 4) // 2, steady, 0)

    for b in range(2):
        ci = _NCH - 2 + b
        pltpu.make_async_copy(idx_sl(ci), idx_b[b], sem_i[b]).wait()
        pltpu.make_async_copy(rows_b[b], out_sl(ci), sem_s[b]).wait()
        compute(b)
        pltpu.async_copy(rows_b[b], out_sl(ci), sem_s[b])
    for b in range(2):
        pltpu.make_async_copy(rows_b[b], out_sl(_NCH - 2 + b), sem_s[b]).wait()


_emb = functools.partial(
    pl.kernel,
    mesh=plsc.VectorSubcoreMesh(core_axis_name="c", subcore_axis_name="s"),
    compiler_params=pltpu.CompilerParams(use_tc_tiling_on_sc=False,
                                         needs_layout_passes=False),
    out_type=jax.ShapeDtypeStruct((_TOTAL * _DIM,), jnp.float32),
    scratch_types=[
        pltpu.VMEM((_VOCAB * _DIM,), jnp.float32),
        pltpu.VMEM((_CHUNK * _DIM,), jnp.float32),
        pltpu.VMEM((_CHUNK * _DIM,), jnp.float32),
        pltpu.VMEM((_CHUNK,), jnp.int32),
        pltpu.VMEM((_CHUNK,), jnp.int32),
        pltpu.SemaphoreType.DMA,
        pltpu.SemaphoreType.DMA,
        pltpu.SemaphoreType.DMA,
        pltpu.SemaphoreType.DMA,
    ],
)(_emb_body)


def kernel(x, weight):
    out = _emb(x.reshape(-1), weight.reshape(-1))
    return out.reshape(x.shape[0], x.shape[1], _DIM)


# speedup vs baseline: 1.2964x; 1.2964x over previous
import functools

import jax
import jax.numpy as jnp
from jax import lax
from jax.experimental import pallas as pl
from jax.experimental.pallas import tpu as pltpu
from jax.experimental.pallas import tpu_sc as plsc

_VOCAB = 1000
_DIM = 64
_TOTAL = 16384 * 200
_NC, _NS = 2, 16
_NW = _NC * _NS
_PER_W = _TOTAL // _NW
_IDXW = 80
_KSUB = 5
_NSP = 3
_CHUNK = _KSUB * _IDXW
_NCH = _PER_W // _CHUNK


def _emb_body(x_hbm, w_hbm, out_hbm, table_sh, rows0, rows1,
              idx0, idx1, sem_g, sem_g2, sem_i0, sem_i1, sem_s0, sem_s1):
    cid = lax.axis_index("c")
    sid = lax.axis_index("s")
    wid = sid * _NC + cid
    base = wid * (_PER_W // _IDXW)

    @pl.when(sid == 0)
    def _stage():
        pltpu.sync_copy(w_hbm, table_sh)
    plsc.subcore_barrier()

    rows_b = (rows0, rows1)
    idx_b = (idx0, idx1)
    sem_i = (sem_i0, sem_i1)
    sem_s = (sem_s0, sem_s1)

    def idx_sl(ci):
        return x_hbm.at[pl.ds(base + ci * _KSUB, _KSUB)]

    def out_sl(ci):
        return out_hbm.at[pl.ds((base + ci * _KSUB) * _IDXW, _CHUNK)]

    def do_gathers(b):
        cps = [pltpu.async_copy(
                   (table_sh if j < _NSP else w_hbm).at[idx_b[b].at[j]],
                   rows_b[b].at[pl.ds(j * _IDXW, _IDXW)],
                   sem_g if j < _NSP else sem_g2)
               for j in range(_KSUB)]
        for cp in cps:
            cp.wait()

    h0 = pltpu.async_copy(idx_sl(0), idx0, sem_i0)
    h1 = pltpu.async_copy(idx_sl(1), idx1, sem_i1)
    for b, h in ((0, h0), (1, h1)):
        h.wait()
        do_gathers(b)
        pltpu.async_copy(rows_b[b], out_sl(b), sem_s[b])
        pltpu.async_copy(idx_sl(b + 2), idx_b[b], sem_i[b])

    def steady(k, carry):
        ci2 = 2 + 2 * k
        for b in range(2):
            ci = ci2 + b
            pltpu.make_async_copy(idx_sl(ci), idx_b[b], sem_i[b]).wait()
            pltpu.make_async_copy(rows_b[b], out_sl(ci), sem_s[b]).wait()
            do_gathers(b)
            pltpu.async_copy(rows_b[b], out_sl(ci), sem_s[b])
            pltpu.async_copy(idx_sl(ci + 2), idx_b[b], sem_i[b])
        return carry

    lax.fori_loop(0, (_NCH - 4) // 2, steady, 0)

    for b in range(2):
        ci = _NCH - 2 + b
        pltpu.make_async_copy(idx_sl(ci), idx_b[b], sem_i[b]).wait()
        pltpu.make_async_copy(rows_b[b], out_sl(ci), sem_s[b]).wait()
        do_gathers(b)
        pltpu.async_copy(rows_b[b], out_sl(ci), sem_s[b])
    for b in range(2):
        pltpu.make_async_copy(rows_b[b], out_sl(_NCH - 2 + b), sem_s[b]).wait()


_emb = functools.partial(
    pl.kernel,
    mesh=plsc.VectorSubcoreMesh(core_axis_name="c", subcore_axis_name="s"),
    compiler_params=pltpu.CompilerParams(use_tc_tiling_on_sc=False),
    out_type=jax.ShapeDtypeStruct((_TOTAL, _DIM), jnp.float32),
    scratch_types=[
        pltpu.MemorySpace.VMEM_SHARED((_VOCAB, _DIM), jnp.float32),
        pltpu.VMEM((_CHUNK, _DIM), jnp.float32),
        pltpu.VMEM((_CHUNK, _DIM), jnp.float32),
        pltpu.VMEM((_KSUB, _IDXW), jnp.int32),
        pltpu.VMEM((_KSUB, _IDXW), jnp.int32),
        pltpu.SemaphoreType.DMA,
        pltpu.SemaphoreType.DMA,
        pltpu.SemaphoreType.DMA,
        pltpu.SemaphoreType.DMA,
        pltpu.SemaphoreType.DMA,
        pltpu.SemaphoreType.DMA,
    ],
)(_emb_body)


def kernel(x, weight):
    xf = x.reshape(_TOTAL // _IDXW, _IDXW)
    out = _emb(xf, weight)
    return out.reshape(x.shape[0], x.shape[1], _DIM)


# speedup vs baseline: 1.5081x; 1.1633x over previous
import functools

import jax
import jax.numpy as jnp
from jax import lax
from jax.experimental import pallas as pl
from jax.experimental.pallas import tpu as pltpu
from jax.experimental.pallas import tpu_sc as plsc

_VOCAB = 1000
_DIM = 64
_TOTAL = 16384 * 200
_NC, _NS = 2, 16
_NW = _NC * _NS
_PER_W = _TOTAL // _NW
_IDXW = 128
_KSUB = 4
_NSP = 4
_CHUNK = _KSUB * _IDXW
_NCH = _PER_W // _CHUNK


def _emb_body(x_hbm, w_hbm, out_hbm, table_sh, rows0, rows1,
              idx0, idx1, sem_g, sem_g2, sem_i0, sem_i1, sem_s0, sem_s1):
    cid = lax.axis_index("c")
    sid = lax.axis_index("s")
    wid = sid * _NC + cid
    base = wid * (_PER_W // _IDXW)

    @pl.when(sid == 0)
    def _stage():
        pltpu.sync_copy(w_hbm, table_sh)
    plsc.subcore_barrier()

    rows_b = (rows0, rows1)
    idx_b = (idx0, idx1)
    sem_i = (sem_i0, sem_i1)
    sem_s = (sem_s0, sem_s1)

    def idx_sl(ci):
        return x_hbm.at[pl.ds(base + ci * _KSUB, _KSUB)]

    def out_sl(ci):
        return out_hbm.at[pl.ds((base + ci * _KSUB) * _IDXW, _CHUNK)]

    def do_gathers(b):
        cps = [pltpu.async_copy(
                   (table_sh if j < _NSP else w_hbm).at[idx_b[b].at[j]],
                   rows_b[b].at[pl.ds(j * _IDXW, _IDXW)],
                   sem_g if j < _NSP else sem_g2)
               for j in range(_KSUB)]
        for cp in cps:
            cp.wait()

    h0 = pltpu.async_copy(idx_sl(0), idx0, sem_i0)
    h1 = pltpu.async_copy(idx_sl(1), idx1, sem_i1)
    for b, h in ((0, h0), (1, h1)):
        h.wait()
        do_gathers(b)
        pltpu.async_copy(rows_b[b], out_sl(b), sem_s[b])
        pltpu.async_copy(idx_sl(b + 2), idx_b[b], sem_i[b])

    def steady(k, carry):
        ci2 = 2 + 2 * k
        for b in range(2):
            ci = ci2 + b
            pltpu.make_async_copy(idx_sl(ci), idx_b[b], sem_i[b]).wait()
            pltpu.make_async_copy(rows_b[b], out_sl(ci), sem_s[b]).wait()
            do_gathers(b)
            pltpu.async_copy(rows_b[b], out_sl(ci), sem_s[b])
            pltpu.async_copy(idx_sl(ci + 2), idx_b[b], sem_i[b])
        return carry

    lax.fori_loop(0, (_NCH - 4) // 2, steady, 0)

    for b in range(2):
        ci = _NCH - 2 + b
        pltpu.make_async_copy(idx_sl(ci), idx_b[b], sem_i[b]).wait()
        pltpu.make_async_copy(rows_b[b], out_sl(ci), sem_s[b]).wait()
        do_gathers(b)
        pltpu.async_copy(rows_b[b], out_sl(ci), sem_s[b])
    for b in range(2):
        pltpu.make_async_copy(rows_b[b], out_sl(_NCH - 2 + b), sem_s[b]).wait()


_emb = functools.partial(
    pl.kernel,
    mesh=plsc.VectorSubcoreMesh(core_axis_name="c", subcore_axis_name="s"),
    compiler_params=pltpu.CompilerParams(use_tc_tiling_on_sc=False),
    out_type=jax.ShapeDtypeStruct((_TOTAL, _DIM), jnp.float32),
    scratch_types=[
        pltpu.MemorySpace.VMEM_SHARED((_VOCAB, _DIM), jnp.float32),
        pltpu.VMEM((_CHUNK, _DIM), jnp.float32),
        pltpu.VMEM((_CHUNK, _DIM), jnp.float32),
        pltpu.VMEM((_KSUB, _IDXW), jnp.int32),
        pltpu.VMEM((_KSUB, _IDXW), jnp.int32),
        pltpu.SemaphoreType.DMA,
        pltpu.SemaphoreType.DMA,
        pltpu.SemaphoreType.DMA,
        pltpu.SemaphoreType.DMA,
        pltpu.SemaphoreType.DMA,
        pltpu.SemaphoreType.DMA,
    ],
)(_emb_body)


def kernel(x, weight):
    xf = x.reshape(_TOTAL // _IDXW, _IDXW)
    out = _emb(xf, weight)
    return out.reshape(x.shape[0], x.shape[1], _DIM)


# speedup vs baseline: 1.5112x; 1.0020x over previous
import functools

import jax
import jax.numpy as jnp
from jax import lax
from jax.experimental import pallas as pl
from jax.experimental.pallas import tpu as pltpu
from jax.experimental.pallas import tpu_sc as plsc

_VOCAB = 1000
_DIM = 64
_TOTAL = 16384 * 200
_NC, _NS = 2, 16
_NW = _NC * _NS
_PER_W = _TOTAL // _NW
_IDXW = 128
_KSUB = 4
_NSP = 4
_CHUNK = _KSUB * _IDXW
_NCH = _PER_W // _CHUNK


def _emb_body(x_hbm, w_hbm, out_hbm, table_sh, rows0, rows1,
              idx0, idx1, sem_g, sem_g2, sem_i0, sem_i1, sem_s0, sem_s1):
    cid = lax.axis_index("c")
    sid = lax.axis_index("s")
    wid = sid * _NC + cid
    base = wid * (_PER_W // _IDXW)

    @pl.when(sid == 0)
    def _stage():
        pltpu.sync_copy(w_hbm, table_sh)
    plsc.subcore_barrier()

    rows_b = (rows0, rows1)
    idx_b = (idx0, idx1)
    sem_i = (sem_i0, sem_i1)
    sem_s = (sem_s0, sem_s1)

    def idx_sl(ci):
        return x_hbm.at[pl.ds(base + ci * _KSUB, _KSUB)]

    def out_sl(ci):
        return out_hbm.at[pl.ds((base + ci * _KSUB) * _IDXW, _CHUNK)]

    def do_gathers(b):
        cps = [pltpu.async_copy(
                   (table_sh if j < _NSP else w_hbm).at[idx_b[b].at[j]],
                   rows_b[b].at[pl.ds(j * _IDXW, _IDXW)],
                   sem_g if j < _NSP else sem_g2)
               for j in range(_KSUB)]
        for cp in cps:
            cp.wait()

    h0 = pltpu.async_copy(idx_sl(0), idx0, sem_i0)
    h1 = pltpu.async_copy(idx_sl(1), idx1, sem_i1)
    for b, h in ((0, h0), (1, h1)):
        h.wait()
        do_gathers(b)
        pltpu.async_copy(rows_b[b], out_sl(b), sem_s[b])
        pltpu.async_copy(idx_sl(b + 2), idx_b[b], sem_i[b])

    def steady(k, carry):
        ci2 = 2 + 2 * k
        for b in range(2):
            ci = ci2 + b
            pltpu.make_async_copy(idx_sl(ci), idx_b[b], sem_i[b]).wait()
            pltpu.make_async_copy(rows_b[b], out_sl(ci), sem_s[b]).wait()
            do_gathers(b)
            pltpu.async_copy(rows_b[b], out_sl(ci), sem_s[b])
            pltpu.async_copy(idx_sl(ci + 2), idx_b[b], sem_i[b])
        return carry

    lax.fori_loop(0, (_NCH - 4) // 2, steady, 0)

    for b in range(2):
        ci = _NCH - 2 + b
        pltpu.make_async_copy(idx_sl(ci), idx_b[b], sem_i[b]).wait()
        pltpu.make_async_copy(rows_b[b], out_sl(ci), sem_s[b]).wait()
        do_gathers(b)
        pltpu.async_copy(rows_b[b], out_sl(ci), sem_s[b])
    for b in range(2):
        pltpu.make_async_copy(rows_b[b], out_sl(_NCH - 2 + b), sem_s[b]).wait()


_emb = functools.partial(
    pl.kernel,
    mesh=plsc.VectorSubcoreMesh(core_axis_name="c", subcore_axis_name="s"),
    compiler_params=pltpu.CompilerParams(use_tc_tiling_on_sc=False),
    out_type=jax.ShapeDtypeStruct((_TOTAL, _DIM), jnp.float32),
    scratch_types=[
        pltpu.MemorySpace.VMEM_SHARED((_VOCAB, _DIM), jnp.float32),
        pltpu.VMEM((_CHUNK, _DIM), jnp.float32),
        pltpu.VMEM((_CHUNK, _DIM), jnp.float32),
        pltpu.VMEM((_KSUB, _IDXW), jnp.int32),
        pltpu.VMEM((_KSUB, _IDXW), jnp.int32),
        pltpu.SemaphoreType.DMA,
        pltpu.SemaphoreType.DMA,
        pltpu.SemaphoreType.DMA,
        pltpu.SemaphoreType.DMA,
        pltpu.SemaphoreType.DMA,
        pltpu.SemaphoreType.DMA,
    ],
)(_emb_body)


def kernel(x, weight):
    xf = x.reshape(_TOTAL // _IDXW, _IDXW)
    out = _emb(xf, weight)
    return jnp.multiply(out, jnp.float32(1.0)).reshape(
        x.shape[0], x.shape[1], _DIM)


# speedup vs baseline: 2.4576x; 1.6262x over previous
import functools

import jax
import jax.numpy as jnp
from jax import lax
from jax.experimental import pallas as pl
from jax.experimental.pallas import tpu as pltpu
from jax.experimental.pallas import tpu_sc as plsc

_VOCAB = 1000
_DIM = 64
_PADW = 128
_TOTAL = 16384 * 200
_NC, _NS = 2, 16
_NW = _NC * _NS
_PER_W = _TOTAL // _NW
_IDXW = 100
_KSUB = 4
_CHUNK = _KSUB * _IDXW
_NCH = _PER_W // _CHUNK


def _emb_body(x_hbm, w_hbm, out_hbm, table_sh, rows0, rows1,
              idx0, idx1, sem_g, sem_i0, sem_i1, sem_s0, sem_s1):
    cid = lax.axis_index("c")
    sid = lax.axis_index("s")
    wid = sid * _NC + cid
    base = wid * (_PER_W // _IDXW)

    @pl.when(sid == 0)
    def _stage():
        pltpu.sync_copy(w_hbm, table_sh)
    plsc.subcore_barrier()

    rows_b = (rows0, rows1)
    idx_b = (idx0, idx1)
    sem_i = (sem_i0, sem_i1)
    sem_s = (sem_s0, sem_s1)

    def idx_sl(ci):
        return x_hbm.at[pl.ds(base + ci * _KSUB, _KSUB)]

    def out_sl(ci):
        return out_hbm.at[pl.ds((base + ci * _KSUB) * _IDXW, _CHUNK)]

    def do_gathers(b):
        cps = [pltpu.async_copy(table_sh.at[idx_b[b].at[j]],
                                rows_b[b].at[pl.ds(j * _IDXW, _IDXW)], sem_g)
               for j in range(_KSUB)]
        for cp in cps:
            cp.wait()

    h0 = pltpu.async_copy(idx_sl(0), idx0, sem_i0)
    h1 = pltpu.async_copy(idx_sl(1), idx1, sem_i1)
    for b, h in ((0, h0), (1, h1)):
        h.wait()
        do_gathers(b)
        pltpu.async_copy(rows_b[b], out_sl(b), sem_s[b])
        pltpu.async_copy(idx_sl(b + 2), idx_b[b], sem_i[b])

    def steady(k, carry):
        ci2 = 2 + 2 * k
        for b in range(2):
            ci = ci2 + b
            pltpu.make_async_copy(idx_sl(ci), idx_b[b], sem_i[b]).wait()
            pltpu.make_async_copy(rows_b[b], out_sl(ci), sem_s[b]).wait()
            do_gathers(b)
            pltpu.async_copy(rows_b[b], out_sl(ci), sem_s[b])
            pltpu.async_copy(idx_sl(ci + 2), idx_b[b], sem_i[b])
        return carry

    lax.fori_loop(0, (_NCH - 4) // 2, steady, 0)

    for b in range(2):
        ci = _NCH - 2 + b
        pltpu.make_async_copy(idx_sl(ci), idx_b[b], sem_i[b]).wait()
        pltpu.make_async_copy(rows_b[b], out_sl(ci), sem_s[b]).wait()
        do_gathers(b)
        pltpu.async_copy(rows_b[b], out_sl(ci), sem_s[b])
    for b in range(2):
        pltpu.make_async_copy(rows_b[b], out_sl(_NCH - 2 + b), sem_s[b]).wait()


_emb = functools.partial(
    pl.kernel,
    mesh=plsc.VectorSubcoreMesh(core_axis_name="c", subcore_axis_name="s"),
    compiler_params=pltpu.CompilerParams(use_tc_tiling_on_sc=False),
    out_type=jax.ShapeDtypeStruct((_TOTAL, _PADW), jnp.float32),
    scratch_types=[
        pltpu.MemorySpace.VMEM_SHARED((_VOCAB, _PADW), jnp.float32),
        pltpu.VMEM((_CHUNK, _PADW), jnp.float32),
        pltpu.VMEM((_CHUNK, _PADW), jnp.float32),
        pltpu.VMEM((_KSUB, _IDXW), jnp.int32),
        pltpu.VMEM((_KSUB, _IDXW), jnp.int32),
        pltpu.SemaphoreType.DMA,
        pltpu.SemaphoreType.DMA,
        pltpu.SemaphoreType.DMA,
        pltpu.SemaphoreType.DMA,
        pltpu.SemaphoreType.DMA,
    ],
)(_emb_body)


def kernel(x, weight):
    xf = x.reshape(_TOTAL // _IDXW, _IDXW)
    wp = jnp.pad(weight, ((0, 0), (0, _PADW - _DIM)))
    out = _emb(xf, wp)
    return out.reshape(x.shape[0], x.shape[1], _PADW)[:, :, :_DIM]
